# baseline (speedup 1.0000x reference)
import functools

import jax
import jax.numpy as jnp
from jax import lax
from jax.experimental import pallas as pl
from jax.experimental.pallas import tpu as pltpu
from jax.experimental.pallas import tpu_sc as plsc

N = 10000
E = 320000
F = 128
K = 64
NRI = 2
NRA = 2

_BE = 3200


def _g_body(rbf_t_ref, w_ref, g_ref):
    g_ref[...] = lax.dot_general(rbf_t_ref[...], w_ref[...],
                                 (((0,), (0,)), ((), ())),
                                 preferred_element_type=jnp.float32)


def _compute_g_half(rbf_t, k2f_W, h):
    nblk = (E // 2) // _BE
    return pl.pallas_call(
        _g_body,
        grid=(nblk,),
        in_specs=[
            pl.BlockSpec((K, _BE), lambda i, h=h: (0, i + h * nblk)),
            pl.BlockSpec((K, F), lambda i: (0, 0)),
        ],
        out_specs=pl.BlockSpec((_BE, F), lambda i: (i, 0)),
        out_shape=jax.ShapeDtypeStruct((E // 2, F), jnp.float32),
    )(rbf_t, k2f_W)


_BN = 2000


def _xjf_body(x_ref, w_ref, b_ref, o_ref):
    o_ref[...] = (jnp.dot(x_ref[...], w_ref[...],
                          preferred_element_type=jnp.float32) + b_ref[...])


def _compute_xjf(x, Wj, bj):
    return pl.pallas_call(
        _xjf_body,
        grid=(N // _BN,),
        in_specs=[
            pl.BlockSpec((_BN, F), lambda i: (i, 0)),
            pl.BlockSpec((F, F), lambda i: (0, 0)),
            pl.BlockSpec((1, F), lambda i: (0, 0)),
        ],
        out_specs=pl.BlockSpec((_BN, F), lambda i: (i, 0)),
        out_shape=jax.ShapeDtypeStruct((N, F), jnp.float32),
    )(x, Wj, bj.reshape(1, F))


_C = 64
_EH = E // 2
_NCH = _EH // _C
_NW = 32
_BASE_CH = _NCH // _NW
_EXTRA = _NCH - _BASE_CH * _NW
_NP = 10112
_RPT = _NP // 16
_NB = 3


def _sc_edge_body(eoff, g_hbm, xjf_hbm, idxi_hbm, idxj_hbm, out_hbm,
                  idxi_v, idxj_v, rows_v, g_v, macc,
                  sem_idx, sem_in, sem_sc):
    c = lax.axis_index("c")
    s = lax.axis_index("s")
    wid = c * 16 + s

    zero = jnp.zeros((16,), jnp.float32)

    def zfill(i, carry):
        for j in range(F // 16):
            rows_v[0][i, pl.ds(j * 16, 16)] = zero
        return carry

    lax.fori_loop(0, _C, zfill, 0)
    stripe = s * _RPT
    for kk in range(_RPT // _C):
        pltpu.sync_copy(rows_v[0], macc.at[pl.ds(stripe + kk * _C, _C)])
    rem = _RPT - (_RPT // _C) * _C
    if rem:
        pltpu.sync_copy(rows_v[0].at[pl.ds(0, rem)],
                        macc.at[pl.ds(stripe + (_RPT // _C) * _C, rem)])
    plsc.subcore_barrier()

    start = wid * _BASE_CH + jnp.minimum(wid, _EXTRA)
    nch = jnp.where(wid < _EXTRA, _BASE_CH + 1, _BASE_CH)

    def issue_idx(q, b):
        ei = eoff + (start + q) * _C
        pltpu.async_copy(idxi_hbm.at[pl.ds(ei, _C)], idxi_v[b], sem_idx[b])
        pltpu.async_copy(idxj_hbm.at[pl.ds(ei, _C)], idxj_v[b], sem_idx[b])

    def issue_inputs(q, b):
        eb = (start + q) * _C
        ei = eoff + eb
        pltpu.make_async_copy(idxi_hbm.at[pl.ds(ei, _C)], idxi_v[b],
                              sem_idx[b]).wait()
        pltpu.make_async_copy(idxj_hbm.at[pl.ds(ei, _C)], idxj_v[b],
                              sem_idx[b]).wait()
        pltpu.async_copy(xjf_hbm.at[idxj_v[b]], rows_v[b], sem_in[b])
        pltpu.async_copy(g_hbm.at[pl.ds(eb, _C)], g_v[b], sem_in[b])

    def wait_scatter(b):
        pltpu.make_async_copy(rows_v[b], macc.at[idxi_v[b]], sem_sc[b]).wait()

    def finish(q, b):
        eb = (start + q) * _C
        pltpu.make_async_copy(xjf_hbm.at[idxj_v[b]], rows_v[b],
                              sem_in[b]).wait()
        pltpu.make_async_copy(g_hbm.at[pl.ds(eb, _C)], g_v[b],
                              sem_in[b]).wait()

        @plsc.parallel_loop(0, _C, unroll=4)
        def mul(e):
            for j in range(F // 16):
                sl = pl.ds(j * 16, 16)
                rows_v[b][e, sl] = rows_v[b][e, sl] * g_v[b][e, sl]
        pltpu.async_copy(rows_v[b], macc.at[idxi_v[b]], sem_sc[b], add=True)

    issue_idx(0, 0)
    issue_inputs(0, 0)

    def group(gi, carry):
        for b in range(_NB):
            q = gi * _NB + b
            b2 = (b + 1) % _NB

            @pl.when(jnp.logical_and(q + 1 < nch, q >= 2))
            def _():
                wait_scatter(b2)

            @pl.when(q + 1 < nch)
            def _():
                issue_idx(q + 1, b2)

            @pl.when(q < nch)
            def _():
                finish(q, b)

            @pl.when(q + 1 < nch)
            def _():
                issue_inputs(q + 1, b2)
        return carry

    lax.fori_loop(0, (_BASE_CH + 1 + _NB - 1) // _NB, group, 0)

    for j in range(_NB):
        wait_scatter(j)

    plsc.subcore_barrier()
    pltpu.sync_copy(macc.at[pl.ds(stripe, _RPT)],
                    out_hbm.at[c, pl.ds(stripe, _RPT)])


def _sc_edge(g, xjf, idx_i, idx_j, eoff):
    mesh = plsc.VectorSubcoreMesh(core_axis_name="c", subcore_axis_name="s")
    fn = pl.kernel(
        functools.partial(_sc_edge_body, eoff),
        out_type=jax.ShapeDtypeStruct((2, _NP, F), jnp.float32),
        mesh=mesh,
        scratch_types=[
            [pltpu.VMEM((_C,), jnp.int32) for _ in range(_NB)],
            [pltpu.VMEM((_C,), jnp.int32) for _ in range(_NB)],
            [pltpu.VMEM((_C, F), jnp.float32) for _ in range(_NB)],
            [pltpu.VMEM((_C, F), jnp.float32) for _ in range(_NB)],
            pltpu.VMEM_SHARED((_NP, F), jnp.float32),
            [pltpu.SemaphoreType.DMA for _ in range(_NB)],
            [pltpu.SemaphoreType.DMA for _ in range(_NB)],
            [pltpu.SemaphoreType.DMA for _ in range(_NB)],
        ],
    )
    return fn(g, xjf, idx_i, idx_j)


def _post_body(x_ref, p_ref, q_ref, wi_ref, bi_ref, rw1_ref, rb1_ref, rw2_ref,
               rb2_ref, dw_ref, db_ref, u_ref, aw1_ref, ab1_ref, aw2_ref,
               ab2_ref, o_ref):
    x = x_ref[...]
    m = (jnp.dot(x, wi_ref[...], preferred_element_type=jnp.float32)
         + bi_ref[...] + (p_ref[0] + p_ref[1]) + (q_ref[0] + q_ref[1]))
    for l in range(NRI):
        t = jnp.dot(m, rw1_ref[l], preferred_element_type=jnp.float32) + rb1_ref[l]
        m = m + jnp.dot(t, rw2_ref[l], preferred_element_type=jnp.float32) + rb2_ref[l]
    xo = (u_ref[...] * x
          + jnp.dot(m, dw_ref[...], preferred_element_type=jnp.float32)
          + db_ref[...])
    for l in range(NRA):
        t = jnp.dot(xo, aw1_ref[l], preferred_element_type=jnp.float32) + ab1_ref[l]
        xo = xo + jnp.dot(t, aw2_ref[l], preferred_element_type=jnp.float32) + ab2_ref[l]
    o_ref[...] = xo


def _post(x, parts0, parts1, Wi, bi, Ri_W1, Ri_b1, Ri_W2, Ri_b2, dense_W,
          dense_b, u, Ra_W1, Ra_b1, Ra_W2, Ra_b2):
    full2 = pl.BlockSpec((F, F), lambda i: (0, 0))
    fullb = pl.BlockSpec((1, F), lambda i: (0, 0))
    full3 = pl.BlockSpec((NRI, F, F), lambda i: (0, 0, 0))
    full3b = pl.BlockSpec((NRI, 1, F), lambda i: (0, 0, 0))
    pspec = pl.BlockSpec((2, _BN, F), lambda i: (0, i, 0))
    return pl.pallas_call(
        _post_body,
        grid=(N // _BN,),
        in_specs=[
            pl.BlockSpec((_BN, F), lambda i: (i, 0)),
            pspec, pspec,
            full2, fullb, full3, full3b, full3, full3b,
            full2, fullb, fullb, full3, full3b, full3, full3b,
        ],
        out_specs=pl.BlockSpec((_BN, F), lambda i: (i, 0)),
        out_shape=jax.ShapeDtypeStruct((N, F), jnp.float32),
    )(x, parts0, parts1, Wi, bi.reshape(1, F), Ri_W1,
      Ri_b1.reshape(NRI, 1, F), Ri_W2, Ri_b2.reshape(NRI, 1, F), dense_W,
      dense_b.reshape(1, F), u.reshape(1, F), Ra_W1, Ra_b1.reshape(NRA, 1, F),
      Ra_W2, Ra_b2.reshape(NRA, 1, F))


def kernel(x, rbf, idx_i, idx_j, k2f_W, Wi, bi, Wj, bj, Ri_W1, Ri_b1, Ri_W2,
           Ri_b2, dense_W, dense_b, u, Ra_W1, Ra_b1, Ra_W2, Ra_b2):
    rbf_t = rbf.T
    xjf = _compute_xjf(x, Wj, bj)
    g0 = _compute_g_half(rbf_t, k2f_W, 0)
    parts0 = _sc_edge(g0, xjf, idx_i, idx_j, 0)
    g1 = _compute_g_half(rbf_t, k2f_W, 1)
    parts1 = _sc_edge(g1, xjf, idx_i, idx_j, _EH)
    return _post(x, parts0, parts1, Wi, bi, Ri_W1, Ri_b1, Ri_W2, Ri_b2,
                 dense_W, dense_b, u, Ra_W1, Ra_b1, Ra_W2, Ra_b2)


# speedup vs baseline: 1.3188x; 1.3188x over previous
import functools

import jax
import jax.numpy as jnp
from jax import lax
from jax.experimental import pallas as pl
from jax.experimental.pallas import tpu as pltpu
from jax.experimental.pallas import tpu_sc as plsc

N = 10000
E = 320000
F = 128
K = 64
NRI = 2
NRA = 2

_BE = 3200


def _g_body(rbf_t_ref, w_ref, g_ref):
    g_ref[...] = lax.dot_general(rbf_t_ref[...], w_ref[...],
                                 (((0,), (0,)), ((), ())),
                                 preferred_element_type=jnp.float32)


def _compute_g_half(rbf_t, k2f_W, h):
    nblk = (E // 2) // _BE
    return pl.pallas_call(
        _g_body,
        grid=(nblk,),
        in_specs=[
            pl.BlockSpec((K, _BE), lambda i, h=h: (0, i + h * nblk)),
            pl.BlockSpec((K, F), lambda i: (0, 0)),
        ],
        out_specs=pl.BlockSpec((_BE, F), lambda i: (i, 0)),
        out_shape=jax.ShapeDtypeStruct((E // 2, F), jnp.float32),
    )(rbf_t, k2f_W)


_BN = 2000


def _xjf_body(x_ref, w_ref, b_ref, o_ref):
    o_ref[...] = (jnp.dot(x_ref[...], w_ref[...],
                          preferred_element_type=jnp.float32) + b_ref[...])


def _compute_xjf(x, Wj, bj):
    return pl.pallas_call(
        _xjf_body,
        grid=(N // _BN,),
        in_specs=[
            pl.BlockSpec((_BN, F), lambda i: (i, 0)),
            pl.BlockSpec((F, F), lambda i: (0, 0)),
            pl.BlockSpec((1, F), lambda i: (0, 0)),
        ],
        out_specs=pl.BlockSpec((_BN, F), lambda i: (i, 0)),
        out_shape=jax.ShapeDtypeStruct((N, F), jnp.float32),
    )(x, Wj, bj.reshape(1, F))


_C = 64
_EH = E // 2
_NCH = _EH // _C
_NW = 32
_BASE_CH = _NCH // _NW
_EXTRA = _NCH - _BASE_CH * _NW
_NP = 10112
_RPT = _NP // 16
_NB = 3


def _sc_edge_body(eoff, g_hbm, xjf_hbm, idxi_hbm, idxj_hbm, out_hbm,
                  idxi_v, idxj_v, rows_v, g_v, macc,
                  sem_idx, sem_in, sem_sc):
    c = lax.axis_index("c")
    s = lax.axis_index("s")
    wid = c * 16 + s

    zero = jnp.zeros((16,), jnp.float32)

    def zfill(i, carry):
        for j in range(F // 16):
            rows_v[0][i, pl.ds(j * 16, 16)] = zero
        return carry

    lax.fori_loop(0, _C, zfill, 0)
    stripe = s * _RPT
    for kk in range(_RPT // _C):
        pltpu.sync_copy(rows_v[0], macc.at[pl.ds(stripe + kk * _C, _C)])
    rem = _RPT - (_RPT // _C) * _C
    if rem:
        pltpu.sync_copy(rows_v[0].at[pl.ds(0, rem)],
                        macc.at[pl.ds(stripe + (_RPT // _C) * _C, rem)])
    plsc.subcore_barrier()

    start = wid * _BASE_CH + jnp.minimum(wid, _EXTRA)
    nch = jnp.where(wid < _EXTRA, _BASE_CH + 1, _BASE_CH)

    def issue_idx(q, d):
        ei = eoff + (start + q) * _C
        pltpu.async_copy(idxi_hbm.at[pl.ds(ei, _C)], idxi_v[d], sem_idx[d])
        pltpu.async_copy(idxj_hbm.at[pl.ds(ei, _C)], idxj_v[d], sem_idx[d])

    def issue_inputs(q, b, d):
        eb = (start + q) * _C
        ei = eoff + eb
        pltpu.make_async_copy(idxi_hbm.at[pl.ds(ei, _C)], idxi_v[d],
                              sem_idx[d]).wait()
        pltpu.make_async_copy(idxj_hbm.at[pl.ds(ei, _C)], idxj_v[d],
                              sem_idx[d]).wait()
        pltpu.async_copy(xjf_hbm.at[idxj_v[d]], rows_v[b], sem_in[b])
        pltpu.async_copy(g_hbm.at[pl.ds(eb, _C)], g_v[b], sem_in[b])

    def wait_scatter(b, d):
        pltpu.make_async_copy(rows_v[b], macc.at[idxi_v[d]], sem_sc[b]).wait()

    def finish(q, b, d):
        eb = (start + q) * _C
        pltpu.make_async_copy(xjf_hbm.at[idxj_v[d]], rows_v[b],
                              sem_in[b]).wait()
        pltpu.make_async_copy(g_hbm.at[pl.ds(eb, _C)], g_v[b],
                              sem_in[b]).wait()

        @plsc.parallel_loop(0, _C, unroll=4)
        def mul(e):
            for j in range(F // 16):
                sl = pl.ds(j * 16, 16)
                rows_v[b][e, sl] = rows_v[b][e, sl] * g_v[b][e, sl]
        pltpu.async_copy(rows_v[b], macc.at[idxi_v[d]], sem_sc[b], add=True)

    issue_idx(0, 0)
    issue_idx(1, 1)
    issue_inputs(0, 0, 0)

    def group(gi, carry):
        for u in range(12):
            q = gi * 12 + u
            b = u % _NB
            b2 = (u + 1) % _NB

            @pl.when(jnp.logical_and(q + 1 < nch, q >= 2))
            def _():
                wait_scatter(b2, (u + 2) % 4)

            @pl.when(q + 1 < nch)
            def _():
                issue_inputs(q + 1, b2, (u + 1) % 4)

            @pl.when(q + 2 < nch)
            def _():
                issue_idx(q + 2, (u + 2) % 4)

            @pl.when(q < nch)
            def _():
                finish(q, b, u % 4)
        return carry

    lax.fori_loop(0, (_BASE_CH + 1 + 11) // 12, group, 0)

    for j in range(_NB):
        wait_scatter(j, j)

    plsc.subcore_barrier()
    pltpu.sync_copy(macc.at[pl.ds(stripe, _RPT)],
                    out_hbm.at[c, pl.ds(stripe, _RPT)])


def _sc_edge(g, xjf, idx_i, idx_j, eoff):
    mesh = plsc.VectorSubcoreMesh(core_axis_name="c", subcore_axis_name="s")
    fn = pl.kernel(
        functools.partial(_sc_edge_body, eoff),
        out_type=jax.ShapeDtypeStruct((2, _NP, F), jnp.float32),
        mesh=mesh,
        scratch_types=[
            [pltpu.VMEM((_C,), jnp.int32) for _ in range(4)],
            [pltpu.VMEM((_C,), jnp.int32) for _ in range(4)],
            [pltpu.VMEM((_C, F), jnp.float32) for _ in range(_NB)],
            [pltpu.VMEM((_C, F), jnp.float32) for _ in range(_NB)],
            pltpu.VMEM_SHARED((_NP, F), jnp.float32),
            [pltpu.SemaphoreType.DMA for _ in range(4)],
            [pltpu.SemaphoreType.DMA for _ in range(_NB)],
            [pltpu.SemaphoreType.DMA for _ in range(_NB)],
        ],
    )
    return fn(g, xjf, idx_i, idx_j)


def _post_body(x_ref, p_ref, q_ref, wi_ref, bi_ref, rw1_ref, rb1_ref, rw2_ref,
               rb2_ref, dw_ref, db_ref, u_ref, aw1_ref, ab1_ref, aw2_ref,
               ab2_ref, o_ref):
    x = x_ref[...]
    m = (jnp.dot(x, wi_ref[...], preferred_element_type=jnp.float32)
         + bi_ref[...] + (p_ref[0] + p_ref[1]) + (q_ref[0] + q_ref[1]))
    for l in range(NRI):
        t = jnp.dot(m, rw1_ref[l], preferred_element_type=jnp.float32) + rb1_ref[l]
        m = m + jnp.dot(t, rw2_ref[l], preferred_element_type=jnp.float32) + rb2_ref[l]
    xo = (u_ref[...] * x
          + jnp.dot(m, dw_ref[...], preferred_element_type=jnp.float32)
          + db_ref[...])
    for l in range(NRA):
        t = jnp.dot(xo, aw1_ref[l], preferred_element_type=jnp.float32) + ab1_ref[l]
        xo = xo + jnp.dot(t, aw2_ref[l], preferred_element_type=jnp.float32) + ab2_ref[l]
    o_ref[...] = xo


def _post(x, parts0, parts1, Wi, bi, Ri_W1, Ri_b1, Ri_W2, Ri_b2, dense_W,
          dense_b, u, Ra_W1, Ra_b1, Ra_W2, Ra_b2):
    full2 = pl.BlockSpec((F, F), lambda i: (0, 0))
    fullb = pl.BlockSpec((1, F), lambda i: (0, 0))
    full3 = pl.BlockSpec((NRI, F, F), lambda i: (0, 0, 0))
    full3b = pl.BlockSpec((NRI, 1, F), lambda i: (0, 0, 0))
    pspec = pl.BlockSpec((2, _BN, F), lambda i: (0, i, 0))
    return pl.pallas_call(
        _post_body,
        grid=(N // _BN,),
        in_specs=[
            pl.BlockSpec((_BN, F), lambda i: (i, 0)),
            pspec, pspec,
            full2, fullb, full3, full3b, full3, full3b,
            full2, fullb, fullb, full3, full3b, full3, full3b,
        ],
        out_specs=pl.BlockSpec((_BN, F), lambda i: (i, 0)),
        out_shape=jax.ShapeDtypeStruct((N, F), jnp.float32),
    )(x, parts0, parts1, Wi, bi.reshape(1, F), Ri_W1,
      Ri_b1.reshape(NRI, 1, F), Ri_W2, Ri_b2.reshape(NRI, 1, F), dense_W,
      dense_b.reshape(1, F), u.reshape(1, F), Ra_W1, Ra_b1.reshape(NRA, 1, F),
      Ra_W2, Ra_b2.reshape(NRA, 1, F))


def kernel(x, rbf, idx_i, idx_j, k2f_W, Wi, bi, Wj, bj, Ri_W1, Ri_b1, Ri_W2,
           Ri_b2, dense_W, dense_b, u, Ra_W1, Ra_b1, Ra_W2, Ra_b2):
    rbf_t = rbf.T
    xjf = _compute_xjf(x, Wj, bj)
    g0 = _compute_g_half(rbf_t, k2f_W, 0)
    parts0 = _sc_edge(g0, xjf, idx_i, idx_j, 0)
    g1 = _compute_g_half(rbf_t, k2f_W, 1)
    parts1 = _sc_edge(g1, xjf, idx_i, idx_j, _EH)
    return _post(x, parts0, parts1, Wi, bi, Ri_W1, Ri_b1, Ri_W2, Ri_b2,
                 dense_W, dense_b, u, Ra_W1, Ra_b1, Ra_W2, Ra_b2)
